# Initial kernel scaffold; baseline (speedup 1.0000x reference)
import jax
import jax.numpy as jnp
from jax.experimental import pallas as pl


def kernel(index, hidden_states, gate_w, ws):
    raise NotImplementedError("write your pallas kernel here")


# speedup vs baseline: 1.0410x; 1.0410x over previous
import functools

import jax
import jax.numpy as jnp
from jax import lax
from jax.experimental import pallas as pl
from jax.experimental.pallas import tpu as pltpu
from jax.experimental.pallas import tpu_sc as plsc

L = 2
E = 8
TOPK = 2
H = 1024
I = 2048
T = 2048

TM = 256
P = T * TOPK
R = P + E * TM
NT = R // TM

NC = 2
NS = 16
NW = NC * NS
GCH = 64
CCH = 32


def _router_body(x_ref, gw_ref, ei_ref, cw_ref):
    x = x_ref[...]
    gw = gw_ref[...]
    logits = lax.dot_general(x, gw, (((1,), (1,)), ((), ())),
                             preferred_element_type=jnp.float32)
    col = lax.broadcasted_iota(jnp.int32, logits.shape, 1)
    m0 = jnp.max(logits, axis=1, keepdims=True)
    i0 = jnp.min(jnp.where(logits == m0, col, E), axis=1, keepdims=True)
    masked = jnp.where(col == i0, -jnp.inf, logits)
    m1 = jnp.max(masked, axis=1, keepdims=True)
    i1 = jnp.min(jnp.where(masked == m1, col, E), axis=1, keepdims=True)
    c0 = 1.0 / (1.0 + jnp.exp(m1 - m0))
    ei_ref[...] = jnp.concatenate([i0, i1], axis=1)
    cw_ref[...] = jnp.concatenate([c0, 1.0 - c0], axis=1)


def _run_router(hidden, gw):
    return pl.pallas_call(
        _router_body,
        out_shape=(
            jax.ShapeDtypeStruct((T, TOPK), jnp.int32),
            jax.ShapeDtypeStruct((T, TOPK), jnp.float32),
        ),
    )(hidden, gw)


def _build_dispatch(ei, cw):
    e_flat = ei.reshape(-1)
    c_flat = cw.reshape(-1)
    onehot = (e_flat[:, None] == jnp.arange(E, dtype=jnp.int32)[None, :])
    onehot = onehot.astype(jnp.int32)
    incl = jnp.cumsum(onehot, axis=0)
    counts = incl[-1]
    rank = jnp.take_along_axis(incl, e_flat[:, None], axis=1)[:, 0] - 1
    padded = ((counts + TM - 1) // TM) * TM
    ends_cum = jnp.cumsum(padded)
    offs = ends_cum - padded
    row = offs[e_flat] + rank
    tok = jnp.arange(P, dtype=jnp.int32) // TOPK
    g = jnp.zeros((R,), jnp.int32).at[row].set(tok)
    c_row = jnp.zeros((R,), jnp.float32).at[row].set(c_flat)
    pos = row.reshape(T, TOPK).astype(jnp.int32)
    tile_start = jnp.arange(NT, dtype=jnp.int32) * TM
    texp = (jnp.searchsorted(offs, tile_start, side="right") - 1)
    texp = jnp.clip(texp, 0, E - 1).astype(jnp.int32)
    is_real = (tile_start < ends_cum[texp]).astype(jnp.int32)
    return g, c_row.reshape(R, 1), pos[:, 0], pos[:, 1], texp, is_real


def _sc_gather_body(hid_hbm, g_hbm, xs_hbm, idx_v, rows_v, sem):
    wid = lax.axis_index("s") * NC + lax.axis_index("c")
    base = wid * (R // NW)
    for j in range(R // NW // GCH):
        off = base + j * GCH
        pltpu.sync_copy(g_hbm.at[pl.ds(off, GCH)], idx_v)
        pltpu.async_copy(hid_hbm.at[idx_v], rows_v, sem).wait()
        pltpu.sync_copy(rows_v, xs_hbm.at[pl.ds(off, GCH)])


def _run_sc_gather(hidden, g):
    mesh = plsc.VectorSubcoreMesh(core_axis_name="c", subcore_axis_name="s")
    f = pl.kernel(
        _sc_gather_body,
        out_type=jax.ShapeDtypeStruct((R, H), jnp.float32),
        mesh=mesh,
        scratch_types=[
            pltpu.VMEM((GCH,), jnp.int32),
            pltpu.VMEM((GCH, H), jnp.float32),
            pltpu.SemaphoreType.DMA,
        ],
    )
    return f(hidden, g)


def _ffn_body(idxr, te, ir, x_ref, w1_ref, w3_ref, w2_ref, c_ref, y_ref):
    i = pl.program_id(0)

    @pl.when(ir[i] == 1)
    def _():
        x = x_ref[...]
        h1 = lax.dot_general(x, w1_ref[...], (((1,), (1,)), ((), ())),
                             preferred_element_type=jnp.float32)
        h3 = lax.dot_general(x, w3_ref[...], (((1,), (1,)), ((), ())),
                             preferred_element_type=jnp.float32)
        act = (h1 * jax.nn.sigmoid(h1)) * h3 * c_ref[...]
        y_ref[...] = lax.dot_general(act, w2_ref[...], (((1,), (1,)), ((), ())),
                                     preferred_element_type=jnp.float32)


def _run_ffn(xs, ws_ih, ws_hi, c_row, idx_arr, texp, is_real):
    grid_spec = pltpu.PrefetchScalarGridSpec(
        num_scalar_prefetch=3,
        grid=(NT,),
        in_specs=[
            pl.BlockSpec((TM, H), lambda i, idxr, te, ir: (i, 0)),
            pl.BlockSpec((None, None, None, I, H),
                         lambda i, idxr, te, ir: (idxr[0], te[i], 0, 0, 0)),
            pl.BlockSpec((None, None, None, I, H),
                         lambda i, idxr, te, ir: (idxr[0], te[i], 1, 0, 0)),
            pl.BlockSpec((None, None, None, H, I),
                         lambda i, idxr, te, ir: (idxr[0], te[i], 2, 0, 0)),
            pl.BlockSpec((TM, 1), lambda i, idxr, te, ir: (i, 0)),
        ],
        out_specs=pl.BlockSpec((TM, H), lambda i, idxr, te, ir: (i, 0)),
    )
    return pl.pallas_call(
        _ffn_body,
        grid_spec=grid_spec,
        out_shape=jax.ShapeDtypeStruct((R, H), jnp.float32),
    )(idx_arr, texp, is_real, xs, ws_ih, ws_ih, ws_hi, c_row)


def _sc_combine_body(ys_hbm, p0_hbm, p1_hbm, out_hbm, idx_v, b0, b1, sem):
    wid = lax.axis_index("s") * NC + lax.axis_index("c")
    base = wid * (T // NW)
    for j in range(T // NW // CCH):
        off = base + j * CCH
        pltpu.sync_copy(p0_hbm.at[pl.ds(off, CCH)], idx_v)
        pltpu.async_copy(ys_hbm.at[idx_v], b0, sem).wait()
        pltpu.sync_copy(p1_hbm.at[pl.ds(off, CCH)], idx_v)
        pltpu.async_copy(ys_hbm.at[idx_v], b1, sem).wait()

        def body(r, carry):
            for v in range(H // 16):
                sl = pl.ds(v * 16, 16)
                b0[r, sl] = b0[r, sl] + b1[r, sl]
            return carry

        lax.fori_loop(0, CCH, body, 0)
        pltpu.sync_copy(b0, out_hbm.at[pl.ds(off, CCH)])


def _run_sc_combine(ys, p0, p1):
    mesh = plsc.VectorSubcoreMesh(core_axis_name="c", subcore_axis_name="s")
    f = pl.kernel(
        _sc_combine_body,
        out_type=jax.ShapeDtypeStruct((T, H), jnp.float32),
        mesh=mesh,
        scratch_types=[
            pltpu.VMEM((CCH,), jnp.int32),
            pltpu.VMEM((CCH, H), jnp.float32),
            pltpu.VMEM((CCH, H), jnp.float32),
            pltpu.SemaphoreType.DMA,
        ],
    )
    return f(ys, p0, p1)


def kernel(index, hidden_states, gate_w, ws):
    idx_arr = jnp.asarray(index, jnp.int32).reshape(1)
    gw = gate_w[index]
    ws_ih = ws.reshape(L, E, 3, I, H)
    ws_hi = ws.reshape(L, E, 3, H, I)
    ei, cw = _run_router(hidden_states, gw)
    g, c_row, p0, p1, texp, is_real = _build_dispatch(ei, cw)
    xs = _run_sc_gather(hidden_states, g)
    ys = _run_ffn(xs, ws_ih, ws_hi, c_row, idx_arr, texp, is_real)
    return _run_sc_combine(ys, p0, p1)


# speedup vs baseline: 1.7399x; 1.6713x over previous
import functools

import jax
import jax.numpy as jnp
from jax import lax
from jax.experimental import pallas as pl
from jax.experimental.pallas import tpu as pltpu
from jax.experimental.pallas import tpu_sc as plsc

L = 2
E = 8
TOPK = 2
H = 1024
I = 2048
T = 2048

TM = 256
P = T * TOPK
R = P + E * TM
NT = R // TM

NC = 2
NS = 16
NW = NC * NS
GCH = 64
CCH = 32


def _router_body(x_ref, gw_ref, ei_ref, cw_ref):
    x = x_ref[...]
    gw = gw_ref[...]
    logits = lax.dot_general(x, gw, (((1,), (1,)), ((), ())),
                             preferred_element_type=jnp.float32)
    col = lax.broadcasted_iota(jnp.int32, logits.shape, 1)
    m0 = jnp.max(logits, axis=1, keepdims=True)
    i0 = jnp.min(jnp.where(logits == m0, col, E), axis=1, keepdims=True)
    masked = jnp.where(col == i0, -jnp.inf, logits)
    m1 = jnp.max(masked, axis=1, keepdims=True)
    i1 = jnp.min(jnp.where(masked == m1, col, E), axis=1, keepdims=True)
    c0 = 1.0 / (1.0 + jnp.exp(m1 - m0))
    ei_ref[...] = jnp.concatenate([i0, i1], axis=1)
    cw_ref[...] = jnp.concatenate([c0, 1.0 - c0], axis=1)


def _run_router(hidden, gw):
    return pl.pallas_call(
        _router_body,
        out_shape=(
            jax.ShapeDtypeStruct((T, TOPK), jnp.int32),
            jax.ShapeDtypeStruct((T, TOPK), jnp.float32),
        ),
    )(hidden, gw)


def _build_dispatch(ei, cw):
    e_flat = ei.reshape(-1)
    c_flat = cw.reshape(-1)
    onehot = (e_flat[:, None] == jnp.arange(E, dtype=jnp.int32)[None, :])
    onehot = onehot.astype(jnp.int32)
    incl = jnp.cumsum(onehot, axis=0)
    counts = incl[-1]
    rank = jnp.take_along_axis(incl, e_flat[:, None], axis=1)[:, 0] - 1
    padded = ((counts + TM - 1) // TM) * TM
    ends_cum = jnp.cumsum(padded)
    offs = ends_cum - padded
    row = offs[e_flat] + rank
    tok = jnp.arange(P, dtype=jnp.int32) // TOPK
    g = jnp.zeros((R,), jnp.int32).at[row].set(tok)
    c_row = jnp.zeros((R,), jnp.float32).at[row].set(c_flat)
    pos = row.reshape(T, TOPK).astype(jnp.int32)
    tile_start = jnp.arange(NT, dtype=jnp.int32) * TM
    texp = (jnp.searchsorted(offs, tile_start, side="right") - 1)
    texp = jnp.clip(texp, 0, E - 1).astype(jnp.int32)
    is_real = (tile_start < ends_cum[texp]).astype(jnp.int32)
    return g, c_row.reshape(R, 1), pos[:, 0], pos[:, 1], texp, is_real


def _sc_gather_body(hid_hbm, g_hbm, xs_hbm, idx_v, rows_v, sem):
    wid = lax.axis_index("s") * NC + lax.axis_index("c")
    base = wid * (R // NW)
    for j in range(R // NW // GCH):
        off = base + j * GCH
        pltpu.sync_copy(g_hbm.at[pl.ds(off, GCH)], idx_v)
        pltpu.async_copy(hid_hbm.at[idx_v], rows_v, sem).wait()
        pltpu.sync_copy(rows_v, xs_hbm.at[pl.ds(off, GCH)])


def _run_sc_gather(hidden, g):
    mesh = plsc.VectorSubcoreMesh(core_axis_name="c", subcore_axis_name="s")
    f = pl.kernel(
        _sc_gather_body,
        out_type=jax.ShapeDtypeStruct((R, H), jnp.float32),
        mesh=mesh,
        scratch_types=[
            pltpu.VMEM((GCH,), jnp.int32),
            pltpu.VMEM((GCH, H), jnp.float32),
            pltpu.SemaphoreType.DMA,
        ],
    )
    return f(hidden, g)


def _ffn_body(te, ir, x_ref, w1_ref, w3_ref, w2_ref, c_ref, y_ref):
    i = pl.program_id(0)

    @pl.when(ir[i] == 1)
    def _():
        x = x_ref[...].astype(jnp.bfloat16)
        h1 = lax.dot_general(x, w1_ref[...], (((1,), (1,)), ((), ())),
                             preferred_element_type=jnp.float32)
        h3 = lax.dot_general(x, w3_ref[...], (((1,), (1,)), ((), ())),
                             preferred_element_type=jnp.float32)
        act = (h1 * jax.nn.sigmoid(h1)) * h3 * c_ref[...]
        y_ref[...] = lax.dot_general(act.astype(jnp.bfloat16), w2_ref[...],
                                     (((1,), (1,)), ((), ())),
                                     preferred_element_type=jnp.float32)


def _run_ffn(xs, ws_ih, ws_hi, c_row, texp, is_real):
    grid_spec = pltpu.PrefetchScalarGridSpec(
        num_scalar_prefetch=2,
        grid=(NT,),
        in_specs=[
            pl.BlockSpec((TM, H), lambda i, te, ir: (i, 0)),
            pl.BlockSpec((None, None, I, H),
                         lambda i, te, ir: (te[i], 0, 0, 0)),
            pl.BlockSpec((None, None, I, H),
                         lambda i, te, ir: (te[i], 1, 0, 0)),
            pl.BlockSpec((None, None, H, I),
                         lambda i, te, ir: (te[i], 2, 0, 0)),
            pl.BlockSpec((TM, 1), lambda i, te, ir: (i, 0)),
        ],
        out_specs=pl.BlockSpec((TM, H), lambda i, te, ir: (i, 0)),
    )
    return pl.pallas_call(
        _ffn_body,
        grid_spec=grid_spec,
        out_shape=jax.ShapeDtypeStruct((R, H), jnp.float32),
    )(texp, is_real, xs, ws_ih, ws_ih, ws_hi, c_row)


def _sc_combine_body(ys_hbm, p0_hbm, p1_hbm, out_hbm, idx_v, b0, b1, sem):
    wid = lax.axis_index("s") * NC + lax.axis_index("c")
    base = wid * (T // NW)
    for j in range(T // NW // CCH):
        off = base + j * CCH
        pltpu.sync_copy(p0_hbm.at[pl.ds(off, CCH)], idx_v)
        pltpu.async_copy(ys_hbm.at[idx_v], b0, sem).wait()
        pltpu.sync_copy(p1_hbm.at[pl.ds(off, CCH)], idx_v)
        pltpu.async_copy(ys_hbm.at[idx_v], b1, sem).wait()

        def body(r, carry):
            for v in range(H // 16):
                sl = pl.ds(v * 16, 16)
                b0[r, sl] = b0[r, sl] + b1[r, sl]
            return carry

        lax.fori_loop(0, CCH, body, 0)
        pltpu.sync_copy(b0, out_hbm.at[pl.ds(off, CCH)])


def _run_sc_combine(ys, p0, p1):
    mesh = plsc.VectorSubcoreMesh(core_axis_name="c", subcore_axis_name="s")
    f = pl.kernel(
        _sc_combine_body,
        out_type=jax.ShapeDtypeStruct((T, H), jnp.float32),
        mesh=mesh,
        scratch_types=[
            pltpu.VMEM((CCH,), jnp.int32),
            pltpu.VMEM((CCH, H), jnp.float32),
            pltpu.VMEM((CCH, H), jnp.float32),
            pltpu.SemaphoreType.DMA,
        ],
    )
    return f(ys, p0, p1)


def kernel(index, hidden_states, gate_w, ws):
    gw = gate_w[index]
    ws_l = ws[index]
    ws_ih = ws_l.reshape(E, 3, I, H).astype(jnp.bfloat16)
    ws_hi = ws_l.reshape(E, 3, H, I).astype(jnp.bfloat16)
    ei, cw = _run_router(hidden_states, gw)
    g, c_row, p0, p1, texp, is_real = _build_dispatch(ei, cw)
    xs = _run_sc_gather(hidden_states, g)
    ys = _run_ffn(xs, ws_ih, ws_hi, c_row, texp, is_real)
    return _run_sc_combine(ys, p0, p1)


# speedup vs baseline: 1.7470x; 1.0041x over previous
import functools

import jax
import jax.numpy as jnp
from jax import lax
from jax.experimental import pallas as pl
from jax.experimental.pallas import tpu as pltpu
from jax.experimental.pallas import tpu_sc as plsc

L = 2
E = 8
TOPK = 2
H = 1024
I = 2048
T = 2048

TM = 256
P = T * TOPK
R = P + E * TM
NT = R // TM

NC = 2
NS = 16
NW = NC * NS
GCH = 48
CCH = 32


def _router_body(x_ref, gw_ref, ei_ref, cw_ref):
    x = x_ref[...]
    gw = gw_ref[...]
    logits = lax.dot_general(x, gw, (((1,), (1,)), ((), ())),
                             preferred_element_type=jnp.float32)
    col = lax.broadcasted_iota(jnp.int32, logits.shape, 1)
    m0 = jnp.max(logits, axis=1, keepdims=True)
    i0 = jnp.min(jnp.where(logits == m0, col, E), axis=1, keepdims=True)
    masked = jnp.where(col == i0, -jnp.inf, logits)
    m1 = jnp.max(masked, axis=1, keepdims=True)
    i1 = jnp.min(jnp.where(masked == m1, col, E), axis=1, keepdims=True)
    c0 = 1.0 / (1.0 + jnp.exp(m1 - m0))
    ei_ref[...] = jnp.concatenate([i0, i1], axis=1)
    cw_ref[...] = jnp.concatenate([c0, 1.0 - c0], axis=1)


def _run_router(hidden, gw):
    return pl.pallas_call(
        _router_body,
        out_shape=(
            jax.ShapeDtypeStruct((T, TOPK), jnp.int32),
            jax.ShapeDtypeStruct((T, TOPK), jnp.float32),
        ),
    )(hidden, gw)


def _build_dispatch(ei, cw):
    e_flat = ei.reshape(-1)
    c_flat = cw.reshape(-1)
    onehot = (e_flat[:, None] == jnp.arange(E, dtype=jnp.int32)[None, :])
    onehot = onehot.astype(jnp.int32)
    incl = jnp.cumsum(onehot, axis=0)
    counts = incl[-1]
    rank = jnp.take_along_axis(incl, e_flat[:, None], axis=1)[:, 0] - 1
    padded = ((counts + TM - 1) // TM) * TM
    ends_cum = jnp.cumsum(padded)
    offs = ends_cum - padded
    row = offs[e_flat] + rank
    tok = jnp.arange(P, dtype=jnp.int32) // TOPK
    g = jnp.zeros((R,), jnp.int32).at[row].set(tok)
    c_row = jnp.zeros((R,), jnp.float32).at[row].set(c_flat)
    pos = row.reshape(T, TOPK).astype(jnp.int32)
    tile_start = jnp.arange(NT, dtype=jnp.int32) * TM
    texp = (jnp.searchsorted(offs, tile_start, side="right") - 1)
    texp = jnp.clip(texp, 0, E - 1).astype(jnp.int32)
    is_real = (tile_start < ends_cum[texp]).astype(jnp.int32)
    return g, c_row.reshape(R, 1), pos[:, 0], pos[:, 1], texp, is_real


def _sc_gather_body(hid_hbm, g_hbm, xs_hbm, idx_all, b0, b1,
                    gs0, gs1, ws0, ws1):
    wid = lax.axis_index("s") * NC + lax.axis_index("c")
    rpw = R // NW
    base = wid * rpw
    nch = rpw // GCH
    bufs = (b0, b1)
    gsems = (gs0, gs1)
    wsems = (ws0, ws1)
    pltpu.sync_copy(g_hbm.at[pl.ds(base, rpw)], idx_all)
    gops = [None] * nch
    wops = [None] * nch
    for j in range(nch):
        if j >= 2:
            wops[j - 2].wait()
        gops[j] = pltpu.async_copy(
            hid_hbm.at[idx_all.at[pl.ds(j * GCH, GCH)]], bufs[j % 2],
            gsems[j % 2])
        if j >= 1:
            gops[j - 1].wait()
            wops[j - 1] = pltpu.async_copy(
                bufs[(j - 1) % 2],
                xs_hbm.at[pl.ds(base + (j - 1) * GCH, GCH)],
                wsems[(j - 1) % 2])
    gops[nch - 1].wait()
    wops[nch - 1] = pltpu.async_copy(
        bufs[(nch - 1) % 2],
        xs_hbm.at[pl.ds(base + (nch - 1) * GCH, GCH)],
        wsems[(nch - 1) % 2])
    wops[nch - 2].wait()
    wops[nch - 1].wait()


def _run_sc_gather(hidden, g):
    mesh = plsc.VectorSubcoreMesh(core_axis_name="c", subcore_axis_name="s")
    f = pl.kernel(
        _sc_gather_body,
        out_type=jax.ShapeDtypeStruct((R, H), jnp.float32),
        mesh=mesh,
        scratch_types=[
            pltpu.VMEM((R // NW,), jnp.int32),
            pltpu.VMEM((GCH, H), jnp.float32),
            pltpu.VMEM((GCH, H), jnp.float32),
            pltpu.SemaphoreType.DMA,
            pltpu.SemaphoreType.DMA,
            pltpu.SemaphoreType.DMA,
            pltpu.SemaphoreType.DMA,
        ],
    )
    return f(hidden, g)


def _ffn_body(te, ir, x_ref, w1_ref, w3_ref, w2_ref, c_ref, y_ref):
    i = pl.program_id(0)

    @pl.when(ir[i] == 1)
    def _():
        x = x_ref[...].astype(jnp.bfloat16)
        h1 = lax.dot_general(x, w1_ref[...], (((1,), (1,)), ((), ())),
                             preferred_element_type=jnp.float32)
        h3 = lax.dot_general(x, w3_ref[...], (((1,), (1,)), ((), ())),
                             preferred_element_type=jnp.float32)
        act = (h1 * jax.nn.sigmoid(h1)) * h3 * c_ref[...]
        y_ref[...] = lax.dot_general(act.astype(jnp.bfloat16), w2_ref[...],
                                     (((1,), (1,)), ((), ())),
                                     preferred_element_type=jnp.float32)


def _run_ffn(xs, ws_ih, ws_hi, c_row, texp, is_real):
    grid_spec = pltpu.PrefetchScalarGridSpec(
        num_scalar_prefetch=2,
        grid=(NT,),
        in_specs=[
            pl.BlockSpec((TM, H), lambda i, te, ir: (i, 0)),
            pl.BlockSpec((None, None, I, H),
                         lambda i, te, ir: (te[i], 0, 0, 0)),
            pl.BlockSpec((None, None, I, H),
                         lambda i, te, ir: (te[i], 1, 0, 0)),
            pl.BlockSpec((None, None, H, I),
                         lambda i, te, ir: (te[i], 2, 0, 0)),
            pl.BlockSpec((TM, 1), lambda i, te, ir: (i, 0)),
        ],
        out_specs=pl.BlockSpec((TM, H), lambda i, te, ir: (i, 0)),
    )
    return pl.pallas_call(
        _ffn_body,
        grid_spec=grid_spec,
        out_shape=jax.ShapeDtypeStruct((R, H), jnp.float32),
    )(texp, is_real, xs, ws_ih, ws_ih, ws_hi, c_row)


def _sc_combine_body(ys_hbm, p0_hbm, p1_hbm, out_hbm,
                     i0_all, i1_all, b0, b1, s0, s1, wsem):
    wid = lax.axis_index("s") * NC + lax.axis_index("c")
    tpw = T // NW
    base = wid * tpw
    pltpu.sync_copy(p0_hbm.at[pl.ds(base, tpw)], i0_all)
    pltpu.sync_copy(p1_hbm.at[pl.ds(base, tpw)], i1_all)
    wop = None
    for j in range(tpw // CCH):
        off = base + j * CCH
        if wop is not None:
            wop.wait()
        c0 = pltpu.async_copy(ys_hbm.at[i0_all.at[pl.ds(j * CCH, CCH)]],
                              b0, s0)
        c1 = pltpu.async_copy(ys_hbm.at[i1_all.at[pl.ds(j * CCH, CCH)]],
                              b1, s1)
        c0.wait()
        c1.wait()

        def body(r, carry):
            for v in range(H // 16):
                sl = pl.ds(v * 16, 16)
                b0[r, sl] = b0[r, sl] + b1[r, sl]
            return carry

        lax.fori_loop(0, CCH, body, 0)
        wop = pltpu.async_copy(b0, out_hbm.at[pl.ds(off, CCH)], wsem)
    wop.wait()


def _run_sc_combine(ys, p0, p1):
    mesh = plsc.VectorSubcoreMesh(core_axis_name="c", subcore_axis_name="s")
    f = pl.kernel(
        _sc_combine_body,
        out_type=jax.ShapeDtypeStruct((T, H), jnp.float32),
        mesh=mesh,
        scratch_types=[
            pltpu.VMEM((T // NW,), jnp.int32),
            pltpu.VMEM((T // NW,), jnp.int32),
            pltpu.VMEM((CCH, H), jnp.float32),
            pltpu.VMEM((CCH, H), jnp.float32),
            pltpu.SemaphoreType.DMA,
            pltpu.SemaphoreType.DMA,
            pltpu.SemaphoreType.DMA,
        ],
    )
    return f(ys, p0, p1)


def kernel(index, hidden_states, gate_w, ws):
    gw = gate_w[index]
    ws_l = ws[index]
    ws_ih = ws_l.reshape(E, 3, I, H).astype(jnp.bfloat16)
    ws_hi = ws_l.reshape(E, 3, H, I).astype(jnp.bfloat16)
    ei, cw = _run_router(hidden_states, gw)
    g, c_row, p0, p1, texp, is_real = _build_dispatch(ei, cw)
    xs = _run_sc_gather(hidden_states, g)
    ys = _run_ffn(xs, ws_ih, ws_hi, c_row, texp, is_real)
    return _run_sc_combine(ys, p0, p1)


# speedup vs baseline: 2.1818x; 1.2489x over previous
import functools

import jax
import jax.numpy as jnp
from jax import lax
from jax.experimental import pallas as pl
from jax.experimental.pallas import tpu as pltpu
from jax.experimental.pallas import tpu_sc as plsc

L = 2
E = 8
TOPK = 2
H = 1024
I = 2048
T = 2048

TM = 256
P = T * TOPK
R = P + E * TM
NT = R // TM

NC = 2
NS = 16
NW = NC * NS
GCH = 48
CCH = 32


def _router_body(x_ref, gw_ref, ei_ref, cw_ref):
    x = x_ref[...]
    gw = gw_ref[...]
    logits = lax.dot_general(x, gw, (((1,), (1,)), ((), ())),
                             preferred_element_type=jnp.float32)
    col = lax.broadcasted_iota(jnp.int32, logits.shape, 1)
    m0 = jnp.max(logits, axis=1, keepdims=True)
    i0 = jnp.min(jnp.where(logits == m0, col, E), axis=1, keepdims=True)
    masked = jnp.where(col == i0, -jnp.inf, logits)
    m1 = jnp.max(masked, axis=1, keepdims=True)
    i1 = jnp.min(jnp.where(masked == m1, col, E), axis=1, keepdims=True)
    c0 = 1.0 / (1.0 + jnp.exp(m1 - m0))
    ei_ref[...] = jnp.concatenate([i0, i1], axis=1)
    cw_ref[...] = jnp.concatenate([c0, 1.0 - c0], axis=1)


def _run_router(hidden, gw):
    return pl.pallas_call(
        _router_body,
        out_shape=(
            jax.ShapeDtypeStruct((T, TOPK), jnp.int32),
            jax.ShapeDtypeStruct((T, TOPK), jnp.float32),
        ),
    )(hidden, gw)


def _build_dispatch(ei, cw):
    e_flat = ei.reshape(-1)
    c_flat = cw.reshape(-1)
    onehot = (e_flat[:, None] == jnp.arange(E, dtype=jnp.int32)[None, :])
    onehot = onehot.astype(jnp.int32)
    incl = jnp.cumsum(onehot, axis=0)
    counts = incl[-1]
    rank = jnp.take_along_axis(incl, e_flat[:, None], axis=1)[:, 0] - 1
    padded = ((counts + TM - 1) // TM) * TM
    ends_cum = jnp.cumsum(padded)
    offs = ends_cum - padded
    row = offs[e_flat] + rank
    tok = jnp.arange(P, dtype=jnp.int32) // TOPK
    g = (jnp.arange(R, dtype=jnp.int32) % T).at[row].set(tok)
    c_row = jnp.zeros((R,), jnp.float32).at[row].set(c_flat)
    pos = row.reshape(T, TOPK).astype(jnp.int32)
    tile_start = jnp.arange(NT, dtype=jnp.int32) * TM
    texp = (jnp.searchsorted(offs, tile_start, side="right") - 1)
    texp = jnp.clip(texp, 0, E - 1).astype(jnp.int32)
    is_real = (tile_start < ends_cum[texp]).astype(jnp.int32)
    return g, c_row.reshape(R, 1), pos[:, 0], pos[:, 1], texp, is_real


def _sc_gather_body(hid_hbm, g_hbm, xs_hbm, idx_all, b0, b1,
                    gs0, gs1, ws0, ws1):
    wid = lax.axis_index("s") * NC + lax.axis_index("c")
    rpw = R // NW
    base = wid * rpw
    nch = rpw // GCH
    bufs = (b0, b1)
    gsems = (gs0, gs1)
    wsems = (ws0, ws1)
    pltpu.sync_copy(g_hbm.at[pl.ds(base, rpw)], idx_all)
    gops = [None] * nch
    wops = [None] * nch
    for j in range(nch):
        if j >= 2:
            wops[j - 2].wait()
        gops[j] = pltpu.async_copy(
            hid_hbm.at[idx_all.at[pl.ds(j * GCH, GCH)]], bufs[j % 2],
            gsems[j % 2])
        if j >= 1:
            gops[j - 1].wait()
            wops[j - 1] = pltpu.async_copy(
                bufs[(j - 1) % 2],
                xs_hbm.at[pl.ds(base + (j - 1) * GCH, GCH)],
                wsems[(j - 1) % 2])
    gops[nch - 1].wait()
    wops[nch - 1] = pltpu.async_copy(
        bufs[(nch - 1) % 2],
        xs_hbm.at[pl.ds(base + (nch - 1) * GCH, GCH)],
        wsems[(nch - 1) % 2])
    wops[nch - 2].wait()
    wops[nch - 1].wait()


def _run_sc_gather(hidden, g):
    mesh = plsc.VectorSubcoreMesh(core_axis_name="c", subcore_axis_name="s")
    f = pl.kernel(
        _sc_gather_body,
        out_type=jax.ShapeDtypeStruct((R, H), jnp.float32),
        mesh=mesh,
        scratch_types=[
            pltpu.VMEM((R // NW,), jnp.int32),
            pltpu.VMEM((GCH, H), jnp.float32),
            pltpu.VMEM((GCH, H), jnp.float32),
            pltpu.SemaphoreType.DMA,
            pltpu.SemaphoreType.DMA,
            pltpu.SemaphoreType.DMA,
            pltpu.SemaphoreType.DMA,
        ],
    )
    return f(hidden, g)


def _ffn_body(te, ir, x_ref, w1_ref, w3_ref, w2_ref, c_ref, y_ref):
    i = pl.program_id(0)

    @pl.when(ir[i] == 1)
    def _():
        x = x_ref[...].astype(jnp.bfloat16)
        h1 = lax.dot_general(x, w1_ref[...], (((1,), (1,)), ((), ())),
                             preferred_element_type=jnp.float32)
        h3 = lax.dot_general(x, w3_ref[...], (((1,), (1,)), ((), ())),
                             preferred_element_type=jnp.float32)
        act = (h1 * jax.nn.sigmoid(h1)) * h3 * c_ref[...]
        y_ref[...] = lax.dot_general(act.astype(jnp.bfloat16), w2_ref[...],
                                     (((1,), (1,)), ((), ())),
                                     preferred_element_type=jnp.float32)


def _run_ffn(xs, ws_ih, ws_hi, c_row, texp, is_real):
    grid_spec = pltpu.PrefetchScalarGridSpec(
        num_scalar_prefetch=2,
        grid=(NT,),
        in_specs=[
            pl.BlockSpec((TM, H), lambda i, te, ir: (i, 0)),
            pl.BlockSpec((None, None, I, H),
                         lambda i, te, ir: (te[i], 0, 0, 0)),
            pl.BlockSpec((None, None, I, H),
                         lambda i, te, ir: (te[i], 1, 0, 0)),
            pl.BlockSpec((None, None, H, I),
                         lambda i, te, ir: (te[i], 2, 0, 0)),
            pl.BlockSpec((TM, 1), lambda i, te, ir: (i, 0)),
        ],
        out_specs=pl.BlockSpec((TM, H), lambda i, te, ir: (i, 0)),
    )
    return pl.pallas_call(
        _ffn_body,
        grid_spec=grid_spec,
        out_shape=jax.ShapeDtypeStruct((R, H), jnp.float32),
    )(texp, is_real, xs, ws_ih, ws_ih, ws_hi, c_row)


def _sc_combine_body(ys_hbm, p0_hbm, p1_hbm, out_hbm,
                     i0_all, i1_all, b0, b1, s0, s1, wsem):
    wid = lax.axis_index("s") * NC + lax.axis_index("c")
    tpw = T // NW
    base = wid * tpw
    pltpu.sync_copy(p0_hbm.at[pl.ds(base, tpw)], i0_all)
    pltpu.sync_copy(p1_hbm.at[pl.ds(base, tpw)], i1_all)
    wop = None
    for j in range(tpw // CCH):
        off = base + j * CCH
        if wop is not None:
            wop.wait()
        c0 = pltpu.async_copy(ys_hbm.at[i0_all.at[pl.ds(j * CCH, CCH)]],
                              b0, s0)
        c1 = pltpu.async_copy(ys_hbm.at[i1_all.at[pl.ds(j * CCH, CCH)]],
                              b1, s1)
        c0.wait()
        c1.wait()

        def body(r, carry):
            for v in range(H // 16):
                sl = pl.ds(v * 16, 16)
                b0[r, sl] = b0[r, sl] + b1[r, sl]
            return carry

        lax.fori_loop(0, CCH, body, 0)
        wop = pltpu.async_copy(b0, out_hbm.at[pl.ds(off, CCH)], wsem)
    wop.wait()


def _run_sc_combine(ys, p0, p1):
    mesh = plsc.VectorSubcoreMesh(core_axis_name="c", subcore_axis_name="s")
    f = pl.kernel(
        _sc_combine_body,
        out_type=jax.ShapeDtypeStruct((T, H), jnp.float32),
        mesh=mesh,
        scratch_types=[
            pltpu.VMEM((T // NW,), jnp.int32),
            pltpu.VMEM((T // NW,), jnp.int32),
            pltpu.VMEM((CCH, H), jnp.float32),
            pltpu.VMEM((CCH, H), jnp.float32),
            pltpu.SemaphoreType.DMA,
            pltpu.SemaphoreType.DMA,
            pltpu.SemaphoreType.DMA,
        ],
    )
    return f(ys, p0, p1)


def kernel(index, hidden_states, gate_w, ws):
    gw = gate_w[index]
    ws_l = ws[index]
    ws_ih = ws_l.reshape(E, 3, I, H).astype(jnp.bfloat16)
    ws_hi = ws_l.reshape(E, 3, H, I).astype(jnp.bfloat16)
    ei, cw = _run_router(hidden_states, gw)
    g, c_row, p0, p1, texp, is_real = _build_dispatch(ei, cw)
    xs = _run_sc_gather(hidden_states, g)
    ys = _run_ffn(xs, ws_ih, ws_hi, c_row, texp, is_real)
    return _run_sc_combine(ys, p0, p1)
